# baseline (speedup 1.0000x reference)
import functools

import jax
import jax.numpy as jnp
from jax import lax
from jax.experimental import pallas as pl
from jax.experimental.pallas import tpu as pltpu
from jax.experimental.pallas import tpu_sc as plsc

NCLASS = 1000
NFEAT = 128
BATCH = 16384
LANES = 16
NC = 2
NS = 16
NW = NC * NS
BPW = BATCH // NW
CHUNK = 64
NCHUNK = BPW // CHUNK
LCOLS = 256
LROWS = 128
SPLIT = 896
TCB = 1024
ROW_CHUNKS = [(SPLIT + k * LROWS, min(LROWS, NCLASS - SPLIT - k * LROWS))
              for k in range((NCLASS - SPLIT + LROWS - 1) // LROWS)]
LSTEPS = [(h, ro, rs) for h in range(BPW // LCOLS) for (ro, rs) in ROW_CHUNKS]


def _sc_body(feat_hbm, lt_hbm, y_hbm, centers_hbm, out_hbm,
             y_v, log_bufs, feat_bufs, cent_bufs, out_v,
             sem_l, sem_f, sem_c):
    wid = lax.axis_index("s") * NC + lax.axis_index("c")
    base = wid * BPW

    pltpu.sync_copy(y_hbm.at[pl.ds(base, BPW)], y_v)

    def logit_copy(s):
        h, ro, rs = LSTEPS[s]
        return pltpu.make_async_copy(
            lt_hbm.at[pl.ds(ro, rs), pl.ds(base + h * LCOLS, LCOLS)],
            log_bufs[s % 2].at[pl.ds(0, rs), :],
            sem_l[s % 2],
        )

    def chunk_copies(c):
        b = c % 2
        feat_cp = pltpu.make_async_copy(
            feat_hbm.at[pl.ds(base + c * CHUNK, CHUNK)],
            feat_bufs[b],
            sem_f[b],
        )
        cent_cp = pltpu.make_async_copy(
            centers_hbm.at[y_v.at[pl.ds(c * CHUNK, CHUNK)]],
            cent_bufs[b],
            sem_c[b],
        )
        return feat_cp, cent_cp

    iota16 = lax.iota(jnp.int32, LANES)

    logit_copy(0).start()
    acc_p = jnp.zeros((LANES,), jnp.float32)
    for s in range(len(LSTEPS)):
        if s + 1 < len(LSTEPS):
            logit_copy(s + 1).start()
        logit_copy(s).wait()
        h, ro, rs = LSTEPS[s]
        buf = log_bufs[s % 2]
        for g in range(LCOLS // LANES):
            y16 = y_v[pl.ds(h * LCOLS + g * LANES, LANES)]
            rl = y16 - ro
            valid = (rl >= 0) & (rl < rs)
            rl0 = jnp.where(valid, rl, 0)
            vals = plsc.load_gather(buf, [rl0, g * LANES + iota16])
            acc_p = acc_p + jnp.where(valid, vals, 0.0)

    for cp in chunk_copies(0):
        cp.start()
    acc_sq = jnp.zeros((LANES,), jnp.float32)
    for c in range(NCHUNK):
        if c + 1 < NCHUNK:
            for cp in chunk_copies(c + 1):
                cp.start()
        for cp in chunk_copies(c):
            cp.wait()
        feat_v, cent_v = feat_bufs[c % 2], cent_bufs[c % 2]

        def row_body(r, acc):
            for j in range(NFEAT // LANES):
                d = (feat_v[r, pl.ds(j * LANES, LANES)]
                     - cent_v[r, pl.ds(j * LANES, LANES)])
                acc = acc + d * d
            return acc

        acc_sq = lax.fori_loop(0, CHUNK, row_body, acc_sq)

    out_v[...] = 0.25 * acc_sq - acc_p
    pltpu.sync_copy(out_v, out_hbm.at[wid])


def _tc_body(lt_ref, y_ref, out_ref):
    labels = y_ref[0, :]
    rows = lax.broadcasted_iota(jnp.int32, (SPLIT, TCB), 0)
    mask = rows == labels[None, :]
    picked = jnp.where(mask, lt_ref[...], 0.0)
    out_ref[...] = jnp.sum(picked, axis=0, keepdims=True)


@jax.jit
def _loss(feat2d, logits_t, y_i32, centers):
    mesh = plsc.VectorSubcoreMesh(core_axis_name="c", subcore_axis_name="s")
    partials = pl.kernel(
        _sc_body,
        out_type=jax.ShapeDtypeStruct((NW, LANES), jnp.float32),
        mesh=mesh,
        compiler_params=pltpu.CompilerParams(
            use_tc_tiling_on_sc=True, needs_layout_passes=False),
        scratch_types=[
            pltpu.VMEM((BPW,), jnp.int32),
            [pltpu.VMEM((LROWS, LCOLS), jnp.float32)] * 2,
            [pltpu.VMEM((CHUNK, NFEAT), jnp.float32)] * 2,
            [pltpu.VMEM((CHUNK, NFEAT), jnp.float32)] * 2,
            pltpu.VMEM((LANES,), jnp.float32),
            [pltpu.SemaphoreType.DMA] * 2,
            [pltpu.SemaphoreType.DMA] * 2,
            [pltpu.SemaphoreType.DMA] * 2,
        ],
    )(feat2d, logits_t, y_i32, centers)

    tc_picked = pl.pallas_call(
        _tc_body,
        grid=(BATCH // TCB,),
        in_specs=[
            pl.BlockSpec((SPLIT, TCB), lambda i: (0, i)),
            pl.BlockSpec((1, TCB), lambda i: (0, i)),
        ],
        out_specs=pl.BlockSpec((1, TCB), lambda i: (0, i)),
        out_shape=jax.ShapeDtypeStruct((1, BATCH), jnp.float32),
    )(logits_t, y_i32.reshape(1, BATCH))

    return (jnp.sum(partials) - jnp.sum(tc_picked)) / BATCH


def kernel(feat, logits, y, centers):
    return _loss(feat, logits.T, y.astype(jnp.int32), centers)


# speedup vs baseline: 1.0331x; 1.0331x over previous
import functools

import jax
import jax.numpy as jnp
from jax import lax
from jax.experimental import pallas as pl
from jax.experimental.pallas import tpu as pltpu
from jax.experimental.pallas import tpu_sc as plsc

NCLASS = 1000
NFEAT = 128
BATCH = 16384
LANES = 16
NC = 2
NS = 16
NW = NC * NS
BPW = BATCH // NW
CHUNK = 64
NCHUNK = BPW // CHUNK
LCOLS = 256
LROWS = 128
SPLIT = 864
TCB = 2048
ROW_CHUNKS = [(SPLIT + k * LROWS, min(LROWS, NCLASS - SPLIT - k * LROWS))
              for k in range((NCLASS - SPLIT + LROWS - 1) // LROWS)]
LSTEPS = [(h, ro, rs) for h in range(BPW // LCOLS) for (ro, rs) in ROW_CHUNKS]


def _sc_body(feat_hbm, lt_hbm, y_hbm, centers_hbm, out_hbm,
             y_v, log_bufs, feat_bufs, cent_bufs, out_v,
             sem_l, sem_f, sem_c):
    wid = lax.axis_index("s") * NC + lax.axis_index("c")
    base = wid * BPW

    pltpu.sync_copy(y_hbm.at[pl.ds(base, BPW)], y_v)

    def logit_copy(s):
        h, ro, rs = LSTEPS[s]
        return pltpu.make_async_copy(
            lt_hbm.at[pl.ds(ro, rs), pl.ds(base + h * LCOLS, LCOLS)],
            log_bufs[s % 2].at[pl.ds(0, rs), :],
            sem_l[s % 2],
        )

    def chunk_copies(c):
        b = c % 2
        feat_cp = pltpu.make_async_copy(
            feat_hbm.at[pl.ds(base + c * CHUNK, CHUNK)],
            feat_bufs[b],
            sem_f[b],
        )
        cent_cp = pltpu.make_async_copy(
            centers_hbm.at[y_v.at[pl.ds(c * CHUNK, CHUNK)]],
            cent_bufs[b],
            sem_c[b],
        )
        return feat_cp, cent_cp

    iota16 = lax.iota(jnp.int32, LANES)

    logit_copy(0).start()
    acc_p = jnp.zeros((LANES,), jnp.float32)
    for s in range(len(LSTEPS)):
        if s + 1 < len(LSTEPS):
            logit_copy(s + 1).start()
        logit_copy(s).wait()
        h, ro, rs = LSTEPS[s]
        buf = log_bufs[s % 2]
        for g in range(LCOLS // LANES):
            y16 = y_v[pl.ds(h * LCOLS + g * LANES, LANES)]
            rl = y16 - ro
            valid = (rl >= 0) & (rl < rs)
            rl0 = jnp.where(valid, rl, 0)
            vals = plsc.load_gather(buf, [rl0, g * LANES + iota16])
            acc_p = acc_p + jnp.where(valid, vals, 0.0)

    for cp in chunk_copies(0):
        cp.start()
    acc_sq = jnp.zeros((LANES,), jnp.float32)
    for c in range(NCHUNK):
        if c + 1 < NCHUNK:
            for cp in chunk_copies(c + 1):
                cp.start()
        for cp in chunk_copies(c):
            cp.wait()
        feat_v, cent_v = feat_bufs[c % 2], cent_bufs[c % 2]

        def row_body(r, acc):
            for j in range(NFEAT // LANES):
                d = (feat_v[r, pl.ds(j * LANES, LANES)]
                     - cent_v[r, pl.ds(j * LANES, LANES)])
                acc = acc + d * d
            return acc

        acc_sq = lax.fori_loop(0, CHUNK, row_body, acc_sq)

    out_v[...] = 0.25 * acc_sq - acc_p
    pltpu.sync_copy(out_v, out_hbm.at[wid])


def _tc_body(lt_ref, y_ref, out_ref):
    labels = y_ref[0, :]
    rows = lax.broadcasted_iota(jnp.int32, (SPLIT, TCB), 0)
    mask = rows == labels[None, :]
    picked = jnp.where(mask, lt_ref[...], 0.0)
    out_ref[...] = jnp.sum(picked, axis=0, keepdims=True)


@jax.jit
def _loss(feat2d, logits_t, y_i32, centers):
    mesh = plsc.VectorSubcoreMesh(core_axis_name="c", subcore_axis_name="s")
    partials = pl.kernel(
        _sc_body,
        out_type=jax.ShapeDtypeStruct((NW, LANES), jnp.float32),
        mesh=mesh,
        compiler_params=pltpu.CompilerParams(
            use_tc_tiling_on_sc=True, needs_layout_passes=False),
        scratch_types=[
            pltpu.VMEM((BPW,), jnp.int32),
            [pltpu.VMEM((LROWS, LCOLS), jnp.float32)] * 2,
            [pltpu.VMEM((CHUNK, NFEAT), jnp.float32)] * 2,
            [pltpu.VMEM((CHUNK, NFEAT), jnp.float32)] * 2,
            pltpu.VMEM((LANES,), jnp.float32),
            [pltpu.SemaphoreType.DMA] * 2,
            [pltpu.SemaphoreType.DMA] * 2,
            [pltpu.SemaphoreType.DMA] * 2,
        ],
    )(feat2d, logits_t, y_i32, centers)

    tc_picked = pl.pallas_call(
        _tc_body,
        grid=(BATCH // TCB,),
        in_specs=[
            pl.BlockSpec((SPLIT, TCB), lambda i: (0, i)),
            pl.BlockSpec((1, TCB), lambda i: (0, i)),
        ],
        out_specs=pl.BlockSpec((1, TCB), lambda i: (0, i)),
        out_shape=jax.ShapeDtypeStruct((1, BATCH), jnp.float32),
    )(logits_t, y_i32.reshape(1, BATCH))

    return (jnp.sum(partials) - jnp.sum(tc_picked)) / BATCH


def kernel(feat, logits, y, centers):
    return _loss(feat, logits.T, y.astype(jnp.int32), centers)


# speedup vs baseline: 1.0345x; 1.0014x over previous
import jax
import jax.numpy as jnp
from jax import lax
from jax.experimental import pallas as pl
from jax.experimental.pallas import tpu as pltpu
from jax.experimental.pallas import tpu_sc as plsc

NCLASS = 1000
NFEAT = 128
BATCH = 16384
LANES = 16
NC = 2
NS = 16
NW = NC * NS
BPW = BATCH // NW
CHUNK = 64
NCHUNK = BPW // CHUNK
LCOLS = 256
LROWS = 128
SPLIT = 896
TCB = 2048
ROW_CHUNKS = [(SPLIT + k * LROWS, min(LROWS, NCLASS - SPLIT - k * LROWS))
              for k in range((NCLASS - SPLIT + LROWS - 1) // LROWS)]
LSTEPS = [(h, ro, rs) for h in range(BPW // LCOLS) for (ro, rs) in ROW_CHUNKS]


def _sc_body(feat_hbm, lt_hbm, y_hbm, centers_hbm, out_hbm,
             y_v, log_bufs, feat_bufs, cent_bufs, out_v,
             sem_l, sem_f, sem_c):
    wid = lax.axis_index("s") * NC + lax.axis_index("c")
    base = wid * BPW

    pltpu.sync_copy(y_hbm.at[pl.ds(base, BPW)], y_v)

    def logit_copy(s):
        h, ro, rs = LSTEPS[s]
        return pltpu.make_async_copy(
            lt_hbm.at[pl.ds(ro, rs), pl.ds(base + h * LCOLS, LCOLS)],
            log_bufs[s % 2].at[pl.ds(0, rs), :],
            sem_l[s % 2],
        )

    def chunk_copies(c):
        b = c % 2
        feat_cp = pltpu.make_async_copy(
            feat_hbm.at[pl.ds(base + c * CHUNK, CHUNK)],
            feat_bufs[b],
            sem_f[b],
        )
        cent_cp = pltpu.make_async_copy(
            centers_hbm.at[y_v.at[pl.ds(c * CHUNK, CHUNK)]],
            cent_bufs[b],
            sem_c[b],
        )
        return feat_cp, cent_cp

    iota16 = lax.iota(jnp.int32, LANES)

    logit_copy(0).start()
    acc_p = jnp.zeros((LANES,), jnp.float32)
    for s in range(len(LSTEPS)):
        if s + 1 < len(LSTEPS):
            logit_copy(s + 1).start()
        logit_copy(s).wait()
        h, ro, rs = LSTEPS[s]
        buf = log_bufs[s % 2]
        for g in range(LCOLS // LANES):
            y16 = y_v[pl.ds(h * LCOLS + g * LANES, LANES)]
            rl = y16 - ro
            valid = (rl >= 0) & (rl < rs)
            rl0 = jnp.where(valid, rl, 0)
            vals = plsc.load_gather(buf, [rl0, g * LANES + iota16])
            acc_p = acc_p + jnp.where(valid, vals, 0.0)

    for cp in chunk_copies(0):
        cp.start()
    acc_sq = jnp.zeros((LANES,), jnp.float32)
    for c in range(NCHUNK):
        if c + 1 < NCHUNK:
            for cp in chunk_copies(c + 1):
                cp.start()
        for cp in chunk_copies(c):
            cp.wait()
        feat_v, cent_v = feat_bufs[c % 2], cent_bufs[c % 2]

        def row_body(r, acc):
            for j in range(NFEAT // LANES):
                d = (feat_v[r, pl.ds(j * LANES, LANES)]
                     - cent_v[r, pl.ds(j * LANES, LANES)])
                acc = acc + d * d
            return acc

        acc_sq = lax.fori_loop(0, CHUNK, row_body, acc_sq)

    out_v[...] = 0.25 * acc_sq - acc_p
    pltpu.sync_copy(out_v, out_hbm.at[wid])


def _tc_body(lt_ref, y_ref, out_ref):
    labels = y_ref[0, :]
    rows = lax.broadcasted_iota(jnp.int32, (SPLIT, TCB), 0)
    mask = rows == labels[None, :]
    picked = jnp.where(mask, lt_ref[...], 0.0)
    out_ref[...] = jnp.sum(picked, axis=0, keepdims=True)


@jax.jit
def _loss(feat2d, logits_t, y_i32, centers):
    mesh = plsc.VectorSubcoreMesh(core_axis_name="c", subcore_axis_name="s")
    partials = pl.kernel(
        _sc_body,
        out_type=jax.ShapeDtypeStruct((NW, LANES), jnp.float32),
        mesh=mesh,
        compiler_params=pltpu.CompilerParams(
            use_tc_tiling_on_sc=True, needs_layout_passes=False),
        scratch_types=[
            pltpu.VMEM((BPW,), jnp.int32),
            [pltpu.VMEM((LROWS, LCOLS), jnp.float32)] * 2,
            [pltpu.VMEM((CHUNK, NFEAT), jnp.float32)] * 2,
            [pltpu.VMEM((CHUNK, NFEAT), jnp.float32)] * 2,
            pltpu.VMEM((LANES,), jnp.float32),
            [pltpu.SemaphoreType.DMA] * 2,
            [pltpu.SemaphoreType.DMA] * 2,
            [pltpu.SemaphoreType.DMA] * 2,
        ],
    )(feat2d, logits_t, y_i32, centers)

    tc_picked = pl.pallas_call(
        _tc_body,
        grid=(BATCH // TCB,),
        in_specs=[
            pl.BlockSpec((SPLIT, TCB), lambda i: (0, i)),
            pl.BlockSpec((1, TCB), lambda i: (0, i)),
        ],
        out_specs=pl.BlockSpec((1, TCB), lambda i: (0, i)),
        out_shape=jax.ShapeDtypeStruct((1, BATCH), jnp.float32),
    )(logits_t, y_i32.reshape(1, BATCH))

    return (jnp.sum(partials) - jnp.sum(tc_picked)) / BATCH


def kernel(feat, logits, y, centers):
    return _loss(feat, logits.T, y.astype(jnp.int32), centers)
